# baseline (speedup 1.0000x reference)
import functools

import jax
import jax.numpy as jnp
from jax import lax
from jax.experimental import pallas as pl
from jax.experimental.pallas import tpu as pltpu
from jax.experimental.pallas import tpu_sc as plsc

EMB_DIM = 32
BATCH = 16384
HALF = BATCH // 2
NC, NS = 2, 16
NW = NC * NS
BPW = HALF // NW
LANES = 128
GROUPS = BPW // 16
NSLOT = 4
RPW = BPW // LANES
WPT = 8 // RPW

_MESH = plsc.VectorSubcoreMesh(
    core_axis_name="c", subcore_axis_name="s", num_cores=NC, num_subcores=NS)


def _sc_gather_body(ug_hbm, ig_hbm, um_hbm, im_hbm, uid_hbm, iid_hbm,
                    out_ug, out_ig, out_um, out_im,
                    uidx_v, iidx_v, ring_ug, ring_ig, ring_um, ring_im,
                    st_ug, st_ig, st_um, st_im,
                    sem_ug, sem_ig, sem_um, sem_im, sem_wb):
    wid = lax.axis_index("s") * NC + lax.axis_index("c")
    tile0 = pl.multiple_of(8 * (wid // WPT), 8)
    pltpu.sync_copy(uid_hbm.at[pl.ds(tile0, 8)], uidx_v)
    pltpu.sync_copy(iid_hbm.at[pl.ds(tile0, 8)], iidx_v)
    row0 = RPW * (wid % WPT)
    iota = lax.iota(jnp.int32, 16)
    tabs = ((ug_hbm, ring_ug, st_ug, sem_ug, 0),
            (ig_hbm, ring_ig, st_ig, sem_ig, 1),
            (um_hbm, ring_um, st_um, sem_um, 0),
            (im_hbm, ring_im, st_im, sem_im, 1))

    def fire(rv, b, slot):
        for hbm, ring, _, sem, which in tabs:
            r = rv[which][b]
            tcol = pl.multiple_of((r // LANES) * LANES, LANES)
            pltpu.async_copy(hbm.at[:, pl.ds(tcol, LANES)],
                             ring.at[slot], sem.at[slot])

    def drain(slot):
        for hbm, ring, _, sem, _w in tabs:
            pltpu.make_async_copy(hbm.at[:, pl.ds(0, LANES)],
                                  ring.at[slot], sem.at[slot]).wait()

    def extract(rv, b, slot, j):
        col = jnp.full((16,), j % 32, jnp.int32)
        buf = (j // 32) % 2
        for hbm, ring, st, sem, which in tabs:
            lane = jnp.full((16,), rv[which][b] % LANES, jnp.int32)
            v0 = plsc.load_gather(ring.at[slot], [iota, lane])
            v1 = plsc.load_gather(ring.at[slot], [iota + 16, lane])
            plsc.store_scatter(st.at[buf], [col, iota], v0)
            plsc.store_scatter(st.at[buf], [col, iota + 16], v1)

    def writeback(block):
        buf = block % 2
        base = pl.multiple_of(BPW * wid + 32 * block, 32)
        pltpu.async_copy(st_ug.at[buf], out_ug.at[pl.ds(base, 32)],
                         sem_wb.at[buf])
        pltpu.async_copy(st_ig.at[buf], out_ig.at[pl.ds(base, 32)],
                         sem_wb.at[buf])
        pltpu.async_copy(st_um.at[buf], out_um.at[pl.ds(base, 32)],
                         sem_wb.at[buf])
        pltpu.async_copy(st_im.at[buf], out_im.at[pl.ds(base, 32)],
                         sem_wb.at[buf])

    def drain_wb(buf):
        for st, out in ((st_ug, out_ug), (st_ig, out_ig),
                        (st_um, out_um), (st_im, out_im)):
            pltpu.make_async_copy(st.at[buf], out.at[pl.ds(0, 32)],
                                  sem_wb.at[buf]).wait()

    def group(g, carry):
        @pl.when(jnp.logical_and(g % 2 == 0, g >= 4))
        def _():
            drain_wb((g // 2) % 2)

        rcur = (plsc.load_gather(uidx_v, [jnp.full((16,), row0 + g // 8,
                                                   jnp.int32),
                                          iota + 16 * (g % 8)]),
                plsc.load_gather(iidx_v, [jnp.full((16,), row0 + g // 8,
                                                   jnp.int32),
                                          iota + 16 * (g % 8)]))
        for b in range(NSLOT):
            @pl.when(g != 0)
            def _(b=b):
                drain(b)
                extract(carry, 12 + b, b, 16 * g + b - 4)
            fire(rcur, b, b)
        for b in range(NSLOT, 16):
            slot = b % NSLOT
            drain(slot)
            extract(rcur, b - 4, slot, 16 * g + b - 4)
            fire(rcur, b, slot)
        @pl.when(jnp.logical_and(g % 2 == 0, g >= 2))
        def _():
            writeback(g // 2 - 1)
        return rcur

    rlast = lax.fori_loop(0, GROUPS, group, (jnp.zeros((16,), jnp.int32),
                                             jnp.zeros((16,), jnp.int32)),
                          unroll=False)
    drain_wb(0)
    for b in range(NSLOT):
        drain(b)
        extract(rlast, 12 + b, b, BPW - 4 + b)
    writeback(BPW // 32 - 1)
    drain_wb(1)


_sc_gather = pl.kernel(
    _sc_gather_body,
    out_type=[jax.ShapeDtypeStruct((HALF, EMB_DIM), jnp.float32)] * 4,
    mesh=_MESH,
    scratch_types=(
        [pltpu.VMEM((8, LANES), jnp.int32)] * 2
        + [pltpu.VMEM((NSLOT, EMB_DIM, LANES), jnp.float32)] * 4
        + [pltpu.VMEM((2, 32, EMB_DIM), jnp.float32)] * 4
        + [pltpu.SemaphoreType.DMA((NSLOT,))] * 4
        + [pltpu.SemaphoreType.DMA((2,))]
    ),
    compiler_params=pltpu.CompilerParams(needs_layout_passes=False),
)


def _mlp_body(ug, ig, um, im, w1u, w1i, b1, w2t, b2, w3t, b3, wfg, wfh, bf,
              out):
    f32 = jnp.float32
    h = jnp.dot(um[...], w1u[...], preferred_element_type=f32)
    h += jnp.dot(im[...], w1i[...], preferred_element_type=f32)
    h = jnp.maximum(h + b1[...], 0.0)
    h = jnp.maximum(jnp.dot(h, w2t[...], preferred_element_type=f32) + b2[...], 0.0)
    h = jnp.maximum(jnp.dot(h, w3t[...], preferred_element_type=f32) + b3[...], 0.0)
    gmf = ug[...] * ig[...]
    logit = (jnp.dot(gmf, wfg[...], preferred_element_type=f32)
             + jnp.dot(h, wfh[...], preferred_element_type=f32) + bf[...])
    out[...] = jax.nn.sigmoid(logit)


_BS = 2048


def _mlp_call(ug, ig, um, im, w1u, w1i, b1, w2t, b2, w3t, b3, wfg, wfh, bf):
    row_spec = pl.BlockSpec((_BS, EMB_DIM), lambda i: (i, 0))
    full = pl.BlockSpec(index_map=lambda i: (0, 0))
    return pl.pallas_call(
        _mlp_body,
        grid=(HALF // _BS,),
        in_specs=[row_spec] * 4 + [full] * 10,
        out_specs=pl.BlockSpec((_BS, 1), lambda i: (i, 0)),
        out_shape=jax.ShapeDtypeStruct((HALF, 1), jnp.float32),
    )(ug, ig, um, im, w1u, w1i, b1, w2t, b2, w3t, b3, wfg, wfh, bf)


def kernel(user_emb_gmf, item_emb_gmf, user_emb_mlp, item_emb_mlp,
           W1, b1, W2, b2, W3, b3, Wf, bf, user_ids, item_ids):
    uid = user_ids.astype(jnp.int32).reshape(2, HALF // LANES, LANES)
    iid = item_ids.astype(jnp.int32).reshape(2, HALF // LANES, LANES)
    tabs = (user_emb_gmf.T, item_emb_gmf.T, user_emb_mlp.T, item_emb_mlp.T)
    w1u = W1[:, :EMB_DIM].T
    w1i = W1[:, EMB_DIM:].T
    wfg = Wf[:, :EMB_DIM].T
    wfh = Wf[:, EMB_DIM:].T
    mlp_w = (w1u, w1i, b1.reshape(1, -1), W2.T, b2.reshape(1, -1), W3.T,
             b3.reshape(1, -1), wfg, wfh, bf.reshape(1, 1))
    outs = [_mlp_call(*_sc_gather(*tabs, uid[h], iid[h]), *mlp_w)
            for h in range(2)]
    return jnp.concatenate(outs, axis=0)


# speedup vs baseline: 1.0087x; 1.0087x over previous
import jax
import jax.numpy as jnp
from jax import lax
from jax.experimental import pallas as pl
from jax.experimental.pallas import tpu as pltpu
from jax.experimental.pallas import tpu_sc as plsc

EMB_DIM = 32
BATCH = 16384
NC, NS = 2, 16
NW = NC * NS
BPW = BATCH // NW
LANES = 128
GROUPS = BPW // 16
NSLOT = 4
IDX2D = (BATCH // LANES, LANES)

_MESH = plsc.VectorSubcoreMesh(
    core_axis_name="c", subcore_axis_name="s", num_cores=NC, num_subcores=NS)


def _sc_gather_body(ug_hbm, ig_hbm, um_hbm, im_hbm, uid_hbm, iid_hbm,
                    out_ug, out_ig, out_um, out_im,
                    uidx_v, iidx_v, ring_ug, ring_ig, ring_um, ring_im,
                    st_ug, st_ig, st_um, st_im,
                    sem_ug, sem_ig, sem_um, sem_im, sem_wb):
    wid = lax.axis_index("s") * NC + lax.axis_index("c")
    tile0 = pl.multiple_of(8 * (wid // 2), 8)
    pltpu.sync_copy(uid_hbm.at[pl.ds(tile0, 8)], uidx_v)
    pltpu.sync_copy(iid_hbm.at[pl.ds(tile0, 8)], iidx_v)
    row0 = 4 * (wid % 2)
    iota = lax.iota(jnp.int32, 16)
    tabs = ((ug_hbm, ring_ug, st_ug, sem_ug, 0),
            (ig_hbm, ring_ig, st_ig, sem_ig, 1),
            (um_hbm, ring_um, st_um, sem_um, 0),
            (im_hbm, ring_im, st_im, sem_im, 1))

    def fire(rv, b, slot):
        for hbm, ring, _, sem, which in tabs:
            r = rv[which][b]
            tcol = pl.multiple_of((r // LANES) * LANES, LANES)
            pltpu.async_copy(hbm.at[:, pl.ds(tcol, LANES)],
                             ring.at[slot], sem.at[slot])

    def drain(slot):
        for hbm, ring, _, sem, _w in tabs:
            pltpu.make_async_copy(hbm.at[:, pl.ds(0, LANES)],
                                  ring.at[slot], sem.at[slot]).wait()

    def extract(rv, b, slot, j):
        col = jnp.full((16,), j % 32, jnp.int32)
        buf = (j // 32) % 2
        for hbm, ring, st, sem, which in tabs:
            lane = jnp.full((16,), rv[which][b] % LANES, jnp.int32)
            v0 = plsc.load_gather(ring.at[slot], [iota, lane])
            v1 = plsc.load_gather(ring.at[slot], [iota + 16, lane])
            plsc.store_scatter(st.at[buf], [col, iota], v0)
            plsc.store_scatter(st.at[buf], [col, iota + 16], v1)

    def writeback(block):
        buf = block % 2
        base = pl.multiple_of(BPW * wid + 32 * block, 32)
        for st, out in ((st_ug, out_ug), (st_ig, out_ig),
                        (st_um, out_um), (st_im, out_im)):
            pltpu.async_copy(st.at[buf], out.at[pl.ds(base, 32)],
                             sem_wb.at[buf])

    def drain_wb(buf):
        for st, out in ((st_ug, out_ug), (st_ig, out_ig),
                        (st_um, out_um), (st_im, out_im)):
            pltpu.make_async_copy(st.at[buf], out.at[pl.ds(0, 32)],
                                  sem_wb.at[buf]).wait()

    def group(g, carry):
        @pl.when(jnp.logical_and(g % 2 == 0, g >= 4))
        def _():
            drain_wb((g // 2) % 2)

        rcur = (plsc.load_gather(uidx_v, [jnp.full((16,), row0 + g // 8,
                                                   jnp.int32),
                                          iota + 16 * (g % 8)]),
                plsc.load_gather(iidx_v, [jnp.full((16,), row0 + g // 8,
                                                   jnp.int32),
                                          iota + 16 * (g % 8)]))
        for b in range(NSLOT):
            @pl.when(g != 0)
            def _(b=b):
                drain(b)
                extract(carry, 12 + b, b, 16 * g + b - 4)
            fire(rcur, b, b)
        for b in range(NSLOT, 16):
            slot = b % NSLOT
            drain(slot)
            extract(rcur, b - 4, slot, 16 * g + b - 4)
            fire(rcur, b, slot)
        @pl.when(jnp.logical_and(g % 2 == 0, g >= 2))
        def _():
            writeback(g // 2 - 1)
        return rcur

    rlast = lax.fori_loop(0, GROUPS, group, (jnp.zeros((16,), jnp.int32),
                                             jnp.zeros((16,), jnp.int32)),
                          unroll=False)
    drain_wb(0)
    for b in range(NSLOT):
        drain(b)
        extract(rlast, 12 + b, b, BPW - 4 + b)
    writeback(15)
    drain_wb(1)


_sc_gather = pl.kernel(
    _sc_gather_body,
    out_type=[jax.ShapeDtypeStruct((BATCH, EMB_DIM), jnp.float32)] * 4,
    mesh=_MESH,
    scratch_types=(
        [pltpu.VMEM((8, LANES), jnp.int32)] * 2
        + [pltpu.VMEM((NSLOT, EMB_DIM, LANES), jnp.float32)] * 4
        + [pltpu.VMEM((2, 32, EMB_DIM), jnp.float32)] * 4
        + [pltpu.SemaphoreType.DMA((NSLOT,))] * 4
        + [pltpu.SemaphoreType.DMA((2,))]
    ),
    compiler_params=pltpu.CompilerParams(needs_layout_passes=False),
)


def _mlp_body(ug, ig, um, im, w1u, w1i, b1, w2t, b2, w3t, b3, wfg, wfh, bf,
              out):
    f32 = jnp.float32
    h = jnp.dot(um[...], w1u[...], preferred_element_type=f32)
    h += jnp.dot(im[...], w1i[...], preferred_element_type=f32)
    h = jnp.maximum(h + b1[...], 0.0)
    h = jnp.maximum(jnp.dot(h, w2t[...], preferred_element_type=f32) + b2[...], 0.0)
    h = jnp.maximum(jnp.dot(h, w3t[...], preferred_element_type=f32) + b3[...], 0.0)
    gmf = ug[...] * ig[...]
    logit = (jnp.dot(gmf, wfg[...], preferred_element_type=f32)
             + jnp.dot(h, wfh[...], preferred_element_type=f32) + bf[...])
    out[...] = jax.nn.sigmoid(logit)


_BS = 2048


def _mlp_call(ug, ig, um, im, w1u, w1i, b1, w2t, b2, w3t, b3, wfg, wfh, bf):
    row_spec = pl.BlockSpec((_BS, EMB_DIM), lambda i: (i, 0))
    full = pl.BlockSpec(index_map=lambda i: (0, 0))
    return pl.pallas_call(
        _mlp_body,
        grid=(BATCH // _BS,),
        in_specs=[row_spec] * 4 + [full] * 10,
        out_specs=pl.BlockSpec((_BS, 1), lambda i: (i, 0)),
        out_shape=jax.ShapeDtypeStruct((BATCH, 1), jnp.float32),
    )(ug, ig, um, im, w1u, w1i, b1, w2t, b2, w3t, b3, wfg, wfh, bf)


def kernel(user_emb_gmf, item_emb_gmf, user_emb_mlp, item_emb_mlp,
           W1, b1, W2, b2, W3, b3, Wf, bf, user_ids, item_ids):
    uid = user_ids.astype(jnp.int32).reshape(IDX2D)
    iid = item_ids.astype(jnp.int32).reshape(IDX2D)
    ug, ig, um, im = _sc_gather(
        user_emb_gmf.T, item_emb_gmf.T, user_emb_mlp.T, item_emb_mlp.T,
        uid, iid)
    w1u = W1[:, :EMB_DIM].T
    w1i = W1[:, EMB_DIM:].T
    wfg = Wf[:, :EMB_DIM].T
    wfh = Wf[:, EMB_DIM:].T
    return _mlp_call(ug, ig, um, im, w1u, w1i, b1.reshape(1, -1),
                     W2.T, b2.reshape(1, -1), W3.T, b3.reshape(1, -1),
                     wfg, wfh, bf.reshape(1, 1))
